# Initial kernel scaffold; baseline (speedup 1.0000x reference)
import jax
import jax.numpy as jnp
from jax.experimental import pallas as pl


def kernel(x, edge_index, W1, b1, W2, b2):
    raise NotImplementedError("write your pallas kernel here")


# speedup vs baseline: 12.9219x; 12.9219x over previous
import functools

import jax
import jax.numpy as jnp
from jax import lax
from jax.experimental import pallas as pl
from jax.experimental.pallas import tpu as pltpu
from jax.experimental.pallas import tpu_sc as plsc

NC = 2
NS = 16
CHUNK = 128
MB = 256


def _sc_edge_scatter(table, src2d, dst2d, npad, d, ch_per_worker):
  rows_per_sub = npad // NS
  n_row_blk = rows_per_sub // CHUNK
  nz = CHUNK * (d // 16)

  def body(table_hbm, src_hbm, dst_hbm, out_hbm, idx_s, idx_d, rows, accum):
    cid = lax.axis_index("c")
    sid = lax.axis_index("s")
    w = sid * NC + cid
    base_r = sid * rows_per_sub

    def zr(i, _):
      rows[i // (d // 16), pl.ds((i % (d // 16)) * 16, 16)] = jnp.zeros(
          (16,), jnp.float32)
      return 0
    lax.fori_loop(0, nz, zr, 0)

    def zb(k, _):
      pltpu.sync_copy(rows, accum.at[pl.ds(base_r + k * CHUNK, CHUNK)])
      return 0
    lax.fori_loop(0, n_row_blk, zb, 0)
    plsc.subcore_barrier()

    pltpu.sync_copy(src_hbm.at[pl.ds(w * ch_per_worker, ch_per_worker)], idx_s)
    pltpu.sync_copy(dst_hbm.at[pl.ds(w * ch_per_worker, ch_per_worker)], idx_d)

    def step(j, _):
      pltpu.sync_copy(table_hbm.at[idx_s.at[j]], rows)
      pltpu.sync_copy(rows, accum.at[idx_d.at[j]], add=True)
      return 0
    lax.fori_loop(0, ch_per_worker, step, 0)
    plsc.subcore_barrier()

    def wb(k, _):
      r0 = base_r + k * CHUNK
      pltpu.sync_copy(accum.at[pl.ds(r0, CHUNK)], rows)
      pltpu.sync_copy(rows, out_hbm.at[cid, pl.ds(r0, CHUNK)])
      return 0
    lax.fori_loop(0, n_row_blk, wb, 0)

  return pl.kernel(
      body,
      out_type=jax.ShapeDtypeStruct((NC, npad, d), jnp.float32),
      mesh=plsc.VectorSubcoreMesh(core_axis_name="c", subcore_axis_name="s"),
      compiler_params=pltpu.CompilerParams(use_tc_tiling_on_sc=False),
      scratch_types=[
          pltpu.VMEM((ch_per_worker, CHUNK), jnp.int32),
          pltpu.VMEM((ch_per_worker, CHUNK), jnp.int32),
          pltpu.VMEM((CHUNK, d), jnp.float32),
          pltpu.VMEM_SHARED((npad, d), jnp.float32),
      ],
  )(table, src2d, dst2d)


def _sc_degree(dst2d, npad, ch_per_worker):
  d = 16
  rows_per_sub = npad // NS
  n_row_blk = rows_per_sub // CHUNK

  def body(dst_hbm, out_hbm, idx_d, rows, accum):
    cid = lax.axis_index("c")
    sid = lax.axis_index("s")
    w = sid * NC + cid
    base_r = sid * rows_per_sub

    def zr(i, _):
      rows[i, pl.ds(0, 16)] = jnp.zeros((16,), jnp.float32)
      return 0
    lax.fori_loop(0, CHUNK, zr, 0)

    def zb(k, _):
      pltpu.sync_copy(rows, accum.at[pl.ds(base_r + k * CHUNK, CHUNK)])
      return 0
    lax.fori_loop(0, n_row_blk, zb, 0)

    def on(i, _):
      rows[i, pl.ds(0, 16)] = jnp.ones((16,), jnp.float32)
      return 0
    lax.fori_loop(0, CHUNK, on, 0)
    plsc.subcore_barrier()

    pltpu.sync_copy(dst_hbm.at[pl.ds(w * ch_per_worker, ch_per_worker)], idx_d)

    def step(j, _):
      pltpu.sync_copy(rows, accum.at[idx_d.at[j]], add=True)
      return 0
    lax.fori_loop(0, ch_per_worker, step, 0)
    plsc.subcore_barrier()

    def wb(k, _):
      r0 = base_r + k * CHUNK
      pltpu.sync_copy(accum.at[pl.ds(r0, CHUNK)], rows)
      pltpu.sync_copy(rows, out_hbm.at[cid, pl.ds(r0, CHUNK)])
      return 0
    lax.fori_loop(0, n_row_blk, wb, 0)

  return pl.kernel(
      body,
      out_type=jax.ShapeDtypeStruct((NC, npad, d), jnp.float32),
      mesh=plsc.VectorSubcoreMesh(core_axis_name="c", subcore_axis_name="s"),
      compiler_params=pltpu.CompilerParams(use_tc_tiling_on_sc=False),
      scratch_types=[
          pltpu.VMEM((ch_per_worker, CHUNK), jnp.int32),
          pltpu.VMEM((CHUNK, d), jnp.float32),
          pltpu.VMEM_SHARED((npad, d), jnp.float32),
      ],
  )(dst2d)


def _dinv_of(dp_ref, mslice):
  deg = dp_ref[0, :, 0:1] + dp_ref[1, :, 0:1] + 1.0
  return lax.rsqrt(deg)


def _tc_layer1(deg_parts, x_pad, w1, npad, f, h):
  def body(dp, xr, w1r, g1):
    dinv = _dinv_of(dp, None)
    g1[...] = dinv * jnp.dot(xr[...], w1r[...],
                             preferred_element_type=jnp.float32)
  return pl.pallas_call(
      body,
      grid=(npad // MB,),
      in_specs=[
          pl.BlockSpec((NC, MB, 16), lambda i: (0, i, 0)),
          pl.BlockSpec((MB, f), lambda i: (i, 0)),
          pl.BlockSpec((f, h), lambda i: (0, 0)),
      ],
      out_specs=pl.BlockSpec((MB, h), lambda i: (i, 0)),
      out_shape=jax.ShapeDtypeStruct((npad, h), jnp.float32),
  )(deg_parts, x_pad, w1)


def _tc_layer2(deg_parts, s1, g1, b1, w2, npad, h, c):
  def body(dp, s1r, g1r, b1r, w2r, g2):
    dinv = _dinv_of(dp, None)
    h1 = dinv * (s1r[0] + s1r[1] + g1r[...]) + b1r[...]
    h1 = jnp.maximum(h1, 0.0)
    g2[...] = dinv * jnp.dot(h1, w2r[...],
                             preferred_element_type=jnp.float32)
  return pl.pallas_call(
      body,
      grid=(npad // MB,),
      in_specs=[
          pl.BlockSpec((NC, MB, 16), lambda i: (0, i, 0)),
          pl.BlockSpec((NC, MB, h), lambda i: (0, i, 0)),
          pl.BlockSpec((MB, h), lambda i: (i, 0)),
          pl.BlockSpec((1, h), lambda i: (0, 0)),
          pl.BlockSpec((h, c), lambda i: (0, 0)),
      ],
      out_specs=pl.BlockSpec((MB, c), lambda i: (i, 0)),
      out_shape=jax.ShapeDtypeStruct((npad, c), jnp.float32),
  )(deg_parts, s1, g1, b1, w2)


def _tc_final(deg_parts, s2, g2, b2, npad, c):
  def body(dp, s2r, g2r, b2r, o):
    dinv = _dinv_of(dp, None)
    o[...] = dinv * (s2r[0] + s2r[1] + g2r[...]) + b2r[...]
  return pl.pallas_call(
      body,
      grid=(npad // MB,),
      in_specs=[
          pl.BlockSpec((NC, MB, 16), lambda i: (0, i, 0)),
          pl.BlockSpec((NC, MB, c), lambda i: (0, i, 0)),
          pl.BlockSpec((MB, c), lambda i: (i, 0)),
          pl.BlockSpec((1, c), lambda i: (0, 0)),
      ],
      out_specs=pl.BlockSpec((MB, c), lambda i: (i, 0)),
      out_shape=jax.ShapeDtypeStruct((npad, c), jnp.float32),
  )(deg_parts, s2, g2, b2)


def kernel(x, edge_index, W1, b1, W2, b2):
  n, f = x.shape
  h = W1.shape[1]
  c = W2.shape[1]
  e = edge_index.shape[1]

  npad = -(-(n + 1) // (NS * CHUNK)) * (NS * CHUNK)
  ch_per_worker = -(-(-(-e // (NC * NS * CHUNK))) // 8) * 8
  erows = ch_per_worker * NC * NS
  epad = erows * CHUNK

  src = edge_index[0]
  dst = edge_index[1]
  pad_idx = jnp.full((epad - e,), n, dtype=jnp.int32)
  src2d = jnp.concatenate([src, pad_idx]).reshape(erows, CHUNK)
  dst2d = jnp.concatenate([dst, pad_idx]).reshape(erows, CHUNK)
  x_pad = jnp.pad(x, ((0, npad - n), (0, 0)))

  deg_parts = _sc_degree(dst2d, npad, ch_per_worker)
  g1 = _tc_layer1(deg_parts, x_pad, W1, npad, f, h)
  s1 = _sc_edge_scatter(g1, src2d, dst2d, npad, h, ch_per_worker)
  g2 = _tc_layer2(deg_parts, s1, g1, b1.reshape(1, h), W2, npad, h, c)
  s2 = _sc_edge_scatter(g2, src2d, dst2d, npad, c, ch_per_worker)
  out = _tc_final(deg_parts, s2, g2, b2.reshape(1, c), npad, c)
  return out[:n]
